# baseline (speedup 1.0000x reference)
import functools

import jax
import jax.numpy as jnp
import numpy as np
from jax import lax
from jax.experimental import pallas as pl
from jax.experimental.pallas import tpu as pltpu
from jax.experimental.pallas import tpu_sc as plsc

_K_TOP = 1024
_I32_MIN = -2147483648
_I32_MAX = 2147483647

_C_NP = np.repeat(np.eye(128, dtype=np.float32), 4, axis=0)


def _gate_body(x_ref, kl_ref, c_ref, g_ref):
    z = jnp.zeros((128, 512), jnp.float32)
    for c in range(3):
        for ky in range(4):
            xb = x_ref[0, c, :, ky * 512:(ky + 1) * 512].astype(jnp.bfloat16).astype(jnp.float32)
            kb = kl_ref[c * 4 + ky:c * 4 + ky + 1, :].astype(jnp.float32)
            z = z + xb * kb
    g_ref[0] = jnp.dot(z, c_ref[...], preferred_element_type=jnp.float32,
                       precision=jax.lax.Precision.HIGHEST)


def _sc_thresh_body(g_hbm, t_hbm, kbuf, hist, tbuf):
    wid = lax.axis_index("s") * 2 + lax.axis_index("c")

    @pl.when(wid < 16)
    def _():
        pltpu.sync_copy(g_hbm.at[wid], kbuf)
        n_vecs = 1024

        def to_keys(i, _):
            v = kbuf[pl.ds(i * 16, 16)]
            b = lax.bitcast_convert_type(v, jnp.int32)
            kbuf[pl.ds(i * 16, 16)] = lax.bitcast_convert_type(
                jnp.where(b >= 0, b, b ^ _I32_MAX), jnp.float32)
            return 0

        lax.fori_loop(0, n_vecs, to_keys, 0)

        jvec = lax.broadcasted_iota(jnp.int32, (16,), 0)
        ones = jnp.ones((16,), jnp.int32)
        lo = jnp.int32(_I32_MIN)
        for sh in (28, 24, 20, 16, 12, 8, 4, 0):
            for zi in range(2):
                hist[pl.ds(zi * 16, 16)] = jnp.zeros((16,), jnp.int32)
            if sh == 28:
                def hpass(i, _):
                    k = lax.bitcast_convert_type(kbuf[pl.ds(i * 16, 16)], jnp.int32)
                    q = (k >> 28) + 8
                    plsc.addupdate_scatter(hist, [q], ones)
                    return 0
            else:
                hi_cl = lo + ((16 << sh) - 1)

                def hpass(i, _, lo=lo, hi_cl=hi_cl, sh=sh):
                    k = lax.bitcast_convert_type(kbuf[pl.ds(i * 16, 16)], jnp.int32)
                    kc = jnp.clip(k, lo, hi_cl)
                    q = (kc - lo) >> sh
                    plsc.addupdate_scatter(hist, [q], ones)
                    return 0

            lax.fori_loop(0, n_vecs, hpass, 0)
            h = hist[pl.ds(0, 16)]
            total = jnp.sum(h)
            cnt_ge = total - (plsc.cumsum(h) - h)
            jstar = jnp.max(jnp.where(cnt_ge >= _K_TOP, jvec, 0))
            lo = lo + (jstar << sh)

        for i in range(8):
            tbuf[pl.ds(i * 16, 16)] = jnp.zeros((16,), jnp.int32) + lo
        pltpu.sync_copy(tbuf, t_hbm.at[wid])


@functools.partial(
    pl.kernel,
    out_type=jax.ShapeDtypeStruct((16, 128), jnp.int32),
    scratch_types=[
        pltpu.VMEM((16384,), jnp.float32),
        pltpu.VMEM((32,), jnp.int32),
        pltpu.VMEM((128,), jnp.int32),
    ],
    mesh=plsc.VectorSubcoreMesh(core_axis_name="c", subcore_axis_name="s"),
    compiler_params=pltpu.CompilerParams(needs_layout_passes=False),
)
def _sc_thresh(g_hbm, t_hbm, kbuf, hist, tbuf):
    _sc_thresh_body(g_hbm, t_hbm, kbuf, hist, tbuf)


def _apply_body(x_ref, g_ref, t_ref, ct_ref, o_ref):
    b = pl.program_id(0)
    g = g_ref[0]
    bits = jax.lax.bitcast_convert_type(g, jnp.int32)
    keys = jnp.where(bits >= 0, bits, bits ^ _I32_MAX)
    trow = t_ref[pl.ds(b, 1), :]
    m = jnp.where(keys >= trow, g, 0.0)
    m_up = jnp.dot(m, ct_ref[...], preferred_element_type=jnp.float32,
                   precision=jax.lax.Precision.HIGHEST)
    for c in range(3):
        for ky in range(4):
            sl = pl.ds(ky * 512, 512)
            o_ref[0, c, :, sl] = x_ref[0, c, :, sl] * m_up


@jax.jit
def _run(x, kl, cmat, ctmat):
    g = pl.pallas_call(
        _gate_body,
        grid=(16,),
        in_specs=[
            pl.BlockSpec((1, 3, 128, 2048), lambda b: (b, 0, 0, 0)),
            pl.BlockSpec((12, 512), lambda b: (0, 0)),
            pl.BlockSpec((512, 128), lambda b: (0, 0)),
        ],
        out_specs=pl.BlockSpec((1, 128, 128), lambda b: (b, 0, 0)),
        out_shape=jax.ShapeDtypeStruct((16, 128, 128), jnp.float32),
        compiler_params=pltpu.CompilerParams(
            dimension_semantics=("arbitrary",),
        ),
    )(x, kl, cmat)

    t = _sc_thresh(g.reshape(16, 16384))

    return pl.pallas_call(
        _apply_body,
        grid=(16,),
        in_specs=[
            pl.BlockSpec((1, 3, 128, 2048), lambda b: (b, 0, 0, 0)),
            pl.BlockSpec((1, 128, 128), lambda b: (b, 0, 0)),
            pl.BlockSpec((16, 128), lambda b: (0, 0)),
            pl.BlockSpec((128, 512), lambda b: (0, 0)),
        ],
        out_specs=pl.BlockSpec((1, 3, 128, 2048), lambda b: (b, 0, 0, 0)),
        out_shape=jax.ShapeDtypeStruct((16, 3, 128, 2048), jnp.float32),
        compiler_params=pltpu.CompilerParams(
            dimension_semantics=("arbitrary",),
        ),
    )(x, g, t, ctmat)


def kernel(inputs, gating_kernel):
    b, cin, H, W = inputs.shape
    x = inputs.reshape(b, cin, 128, 4, 512).reshape(b, cin, 128, 2048)
    w = gating_kernel[0].astype(jnp.bfloat16)
    kl = jnp.tile(w.reshape(12, 1, 4), (1, 128, 1)).reshape(12, 512)
    cmat = jnp.asarray(_C_NP)
    out = _run(x, kl, cmat, cmat.T)
    return out.reshape(b, cin, 128, 4, 512).reshape(b, cin, H, W)


# speedup vs baseline: 1.2407x; 1.2407x over previous
import functools

import jax
import jax.numpy as jnp
import numpy as np
from jax import lax
from jax.experimental import pallas as pl
from jax.experimental.pallas import tpu as pltpu
from jax.experimental.pallas import tpu_sc as plsc

_K_TOP = 1024
_I32_MIN = -2147483648
_I32_MAX = 2147483647

_C_NP = np.repeat(np.eye(128, dtype=np.float32), 4, axis=0)


def _gate_body(x_ref, kl_ref, c_ref, g_ref):
    z = jnp.zeros((128, 512), jnp.float32)
    for c in range(3):
        for ky in range(4):
            xb = x_ref[0, c, :, ky * 512:(ky + 1) * 512].astype(jnp.bfloat16).astype(jnp.float32)
            kb = kl_ref[c * 4 + ky:c * 4 + ky + 1, :].astype(jnp.float32)
            z = z + xb * kb
    g_ref[0] = jnp.dot(z, c_ref[...], preferred_element_type=jnp.float32,
                       precision=jax.lax.Precision.HIGHEST)


def _sc_thresh_body(g_hbm, t_hbm, kbuf, hist, tbuf):
    wid = lax.axis_index("s") * 2 + lax.axis_index("c")

    @pl.when(wid < 16)
    def _():
        pltpu.sync_copy(g_hbm.at[wid], kbuf)
        jvec = lax.broadcasted_iota(jnp.int32, (16,), 0)
        ones = jnp.ones((16,), jnp.int32)
        lo = jnp.int32(_I32_MIN)
        for sh in (24, 16, 8, 0):
            for zi in range(16):
                hist[pl.ds(zi * 16, 16)] = jnp.zeros((16,), jnp.int32)

            if sh == 24:
                def hpass(iv, _):
                    for u in range(8):
                        v = kbuf[pl.ds((iv * 8 + u) * 16, 16)]
                        b = lax.bitcast_convert_type(v, jnp.int32)
                        k = jnp.where(b >= 0, b, b ^ _I32_MAX)
                        q = (k >> 24) + 128
                        plsc.addupdate_scatter(hist, [q], ones)
                    return 0
            else:
                hi_cl = lo + ((256 << sh) - 1)

                def hpass(iv, _, lo=lo, hi_cl=hi_cl, sh=sh):
                    for u in range(8):
                        v = kbuf[pl.ds((iv * 8 + u) * 16, 16)]
                        b = lax.bitcast_convert_type(v, jnp.int32)
                        k = jnp.where(b >= 0, b, b ^ _I32_MAX)
                        kc = jnp.clip(k, lo, hi_cl)
                        q = (kc - lo) >> sh
                        plsc.addupdate_scatter(hist, [q], ones)
                    return 0

            lax.fori_loop(0, 128, hpass, 0)

            acc = jnp.zeros((16,), jnp.int32)
            for v in range(16):
                acc = acc + hist[pl.ds(v * 16, 16)]
            total = jnp.sum(acc)
            carry = jnp.int32(0)
            jstar = jnp.int32(0)
            for v in range(16):
                h = hist[pl.ds(v * 16, 16)]
                pe = carry + (plsc.cumsum(h) - h)
                cnt_ge = total - pe
                jstar = jnp.maximum(
                    jstar,
                    jnp.max(jnp.where(cnt_ge >= _K_TOP, jvec + v * 16, 0)))
                carry = carry + jnp.sum(h)
            lo = lo + (jstar << sh)

        for i in range(8):
            tbuf[pl.ds(i * 16, 16)] = jnp.zeros((16,), jnp.int32) + lo
        pltpu.sync_copy(tbuf, t_hbm.at[wid])


@functools.partial(
    pl.kernel,
    out_type=jax.ShapeDtypeStruct((16, 128), jnp.int32),
    scratch_types=[
        pltpu.VMEM((16384,), jnp.float32),
        pltpu.VMEM((256,), jnp.int32),
        pltpu.VMEM((128,), jnp.int32),
    ],
    mesh=plsc.VectorSubcoreMesh(core_axis_name="c", subcore_axis_name="s"),
    compiler_params=pltpu.CompilerParams(needs_layout_passes=False),
)
def _sc_thresh(g_hbm, t_hbm, kbuf, hist, tbuf):
    _sc_thresh_body(g_hbm, t_hbm, kbuf, hist, tbuf)


def _apply_body(x_ref, g_ref, t_ref, ct_ref, o_ref):
    b = pl.program_id(0)
    g = g_ref[0]
    bits = jax.lax.bitcast_convert_type(g, jnp.int32)
    keys = jnp.where(bits >= 0, bits, bits ^ _I32_MAX)
    trow = t_ref[pl.ds(b, 1), :]
    m = jnp.where(keys >= trow, g, 0.0)
    m_up = jnp.dot(m, ct_ref[...], preferred_element_type=jnp.float32,
                   precision=jax.lax.Precision.HIGHEST)
    for c in range(3):
        for ky in range(4):
            sl = pl.ds(ky * 512, 512)
            o_ref[0, c, :, sl] = x_ref[0, c, :, sl] * m_up


@jax.jit
def _run(x, kl, cmat, ctmat):
    g = pl.pallas_call(
        _gate_body,
        grid=(16,),
        in_specs=[
            pl.BlockSpec((1, 3, 128, 2048), lambda b: (b, 0, 0, 0)),
            pl.BlockSpec((12, 512), lambda b: (0, 0)),
            pl.BlockSpec((512, 128), lambda b: (0, 0)),
        ],
        out_specs=pl.BlockSpec((1, 128, 128), lambda b: (b, 0, 0)),
        out_shape=jax.ShapeDtypeStruct((16, 128, 128), jnp.float32),
        compiler_params=pltpu.CompilerParams(
            dimension_semantics=("arbitrary",),
        ),
    )(x, kl, cmat)

    t = _sc_thresh(g.reshape(16, 16384))

    return pl.pallas_call(
        _apply_body,
        grid=(16,),
        in_specs=[
            pl.BlockSpec((1, 3, 128, 2048), lambda b: (b, 0, 0, 0)),
            pl.BlockSpec((1, 128, 128), lambda b: (b, 0, 0)),
            pl.BlockSpec((16, 128), lambda b: (0, 0)),
            pl.BlockSpec((128, 512), lambda b: (0, 0)),
        ],
        out_specs=pl.BlockSpec((1, 3, 128, 2048), lambda b: (b, 0, 0, 0)),
        out_shape=jax.ShapeDtypeStruct((16, 3, 128, 2048), jnp.float32),
        compiler_params=pltpu.CompilerParams(
            dimension_semantics=("arbitrary",),
        ),
    )(x, g, t, ctmat)


def kernel(inputs, gating_kernel):
    b, cin, H, W = inputs.shape
    x = inputs.reshape(b, cin, 128, 4, 512).reshape(b, cin, 128, 2048)
    w = gating_kernel[0].astype(jnp.bfloat16)
    kl = jnp.tile(w.reshape(12, 1, 4), (1, 128, 1)).reshape(12, 512)
    cmat = jnp.asarray(_C_NP)
    out = _run(x, kl, cmat, cmat.T)
    return out.reshape(b, cin, 128, 4, 512).reshape(b, cin, H, W)
